# baseline (speedup 1.0000x reference)
import dataclasses
import functools

import jax
import jax.numpy as jnp
from jax import lax
from jax.experimental import pallas as pl
from jax.experimental.pallas import tpu as pltpu
from jax.experimental.pallas import tpu_sc as plsc

N = 16384
D_IN = 2048
D_H = 1024
TILE_M = 256
R = 128
NW = 32
ROWS_PW = R // NW
CH = 16
NCH = N // CH
DUMP = N

def _mesh():
    return plsc.VectorSubcoreMesh(core_axis_name="c", subcore_axis_name="s")


def _sc_params():
    cp = pltpu.CompilerParams()
    if "needs_layout_passes" in pltpu.CompilerParams.__dataclass_fields__:
        cp = dataclasses.replace(cp, needs_layout_passes=False)
    return cp


def _route_body(lv_ref, dst_ref, k1_ref, k128_ref):
    lv = lv_ref[...]
    m = (lv == 0).astype(jnp.float32)
    r = lax.broadcasted_iota(jnp.int32, (R, R), 0)
    c = lax.broadcasted_iota(jnp.int32, (R, R), 1)
    upper = (r < c).astype(jnp.float32)
    lower = (c < r).astype(jnp.float32)
    e = lax.dot_general(m, upper, (((1,), (0,)), ((), ())),
                        preferred_element_type=jnp.float32)
    s = e[:, R - 1:R] + m[:, R - 1:R]
    off = lax.dot_general(lower, s, (((1,), (0,)), ((), ())),
                          preferred_element_type=jnp.float32)
    pos = (e + off + 0.5).astype(jnp.int32)
    dst_ref[...] = jnp.where(lv == 0, pos, DUMP + (c % 8))
    k = jnp.sum(m).astype(jnp.int32)
    k1_ref[0] = k
    k128_ref[...] = jnp.full((R,), k, jnp.int32)


def _route(lv2d):
    return pl.pallas_call(
        _route_body,
        out_shape=(
            jax.ShapeDtypeStruct((R, R), jnp.int32),
            jax.ShapeDtypeStruct((1,), jnp.int32),
            jax.ShapeDtypeStruct((R,), jnp.int32),
        ),
        out_specs=(
            pl.BlockSpec((R, R), lambda: (0, 0)),
            pl.BlockSpec(memory_space=pltpu.SMEM),
            pl.BlockSpec((R,), lambda: (0,)),
        ),
    )(lv2d)


def _scatter_idx(dst):
  f = functools.partial(
      pl.kernel,
      out_type=jax.ShapeDtypeStruct((N + 8,), jnp.int32),
      mesh=_mesh(),
      compiler_params=_sc_params(),
      scratch_types=[
          pltpu.VMEM((ROWS_PW, R), jnp.int32),
          pltpu.VMEM((ROWS_PW, R), jnp.int32),
      ],
  )

  @f
  def body(dst_hbm, idx_hbm, dst_v, val_v):
    wid = lax.axis_index("s") * 2 + lax.axis_index("c")
    r0 = wid * ROWS_PW
    pltpu.sync_copy(dst_hbm.at[pl.ds(r0, ROWS_PW)], dst_v)

    @pl.loop(0, ROWS_PW)
    def _rows(rr):
        base = (r0 + rr) * R

        @pl.loop(0, R, step=16)
        def _cols(cs):
            val_v[rr, pl.ds(cs, 16)] = base + cs + lax.iota(jnp.int32, 16)

        pltpu.sync_copy(val_v.at[rr], idx_hbm.at[dst_v.at[rr]])

  return body(dst)


def _gather_rows(idx, k128, x):
  f = functools.partial(
      pl.kernel,
      out_type=jax.ShapeDtypeStruct((N, D_IN), jnp.float32),
      mesh=_mesh(),
      compiler_params=_sc_params(),
      scratch_types=[
          pltpu.VMEM((16,), jnp.int32),
          pltpu.VMEM((CH,), jnp.int32),
          pltpu.VMEM((CH, D_IN), jnp.float32),
      ],
  )

  @f
  def body(idx_hbm, k_hbm, x_hbm, xc_hbm, kv, iv, rows_v):
    wid = lax.axis_index("s") * 2 + lax.axis_index("c")
    pltpu.sync_copy(k_hbm.at[pl.ds(0, 16)], kv)
    k = jnp.max(kv[...], axis=0)

    @pl.loop(0, NCH // NW)
    def _chunks(ci):
        gc = ci * NW + wid

        @pl.when(gc * CH < k)
        def _():
            pltpu.sync_copy(idx_hbm.at[pl.ds(gc * CH, CH)], iv)
            iv[...] = jnp.clip(iv[...], 0, N - 1)
            pltpu.sync_copy(x_hbm.at[iv], rows_v)
            pltpu.sync_copy(rows_v, xc_hbm.at[pl.ds(gc * CH, CH)])

  return body(idx, k128, x)


def _mlp_body(k_ref, x_ref, w1_ref, b1_ref, w2_ref, b2_ref, w3_ref, b3_ref,
              o_ref):
    nt = (k_ref[0] + TILE_M - 1) // TILE_M

    @pl.when(pl.program_id(0) < nt)
    def _():
        xb = x_ref[...].astype(jnp.bfloat16)
        h1 = lax.dot_general(xb, w1_ref[...], (((1,), (0,)), ((), ())),
                             preferred_element_type=jnp.float32)
        h1 = jnp.maximum(h1 + b1_ref[...][None, :], 0.0).astype(jnp.bfloat16)
        h2 = lax.dot_general(h1, w2_ref[...], (((1,), (0,)), ((), ())),
                             preferred_element_type=jnp.float32)
        h2 = jnp.maximum(h2 + b2_ref[...][None, :], 0.0)
        out = lax.dot_general(h2, w3_ref[...], (((1,), (0,)), ((), ())),
                              preferred_element_type=jnp.float32)
        o_ref[...] = out + b3_ref[...][None, :]


def _mlp(k1, xc, w1b, b1, w2b, b2, W3, b3):
    def _x_map(i, kr):
        nt = (kr[0] + TILE_M - 1) // TILE_M
        return (jnp.minimum(i, jnp.maximum(nt - 1, 0)), 0)

    grid_spec = pltpu.PrefetchScalarGridSpec(
        num_scalar_prefetch=1,
        grid=(N // TILE_M,),
        in_specs=[
            pl.BlockSpec((TILE_M, D_IN), _x_map),
            pl.BlockSpec((D_IN, D_H), lambda i, kr: (0, 0)),
            pl.BlockSpec((D_H,), lambda i, kr: (0,)),
            pl.BlockSpec((D_H, D_H), lambda i, kr: (0, 0)),
            pl.BlockSpec((D_H,), lambda i, kr: (0,)),
            pl.BlockSpec((D_H, 1), lambda i, kr: (0, 0)),
            pl.BlockSpec((1,), lambda i, kr: (0,)),
        ],
        out_specs=pl.BlockSpec((TILE_M, 1), lambda i, kr: (i, 0)),
    )
    return pl.pallas_call(
        _mlp_body,
        grid_spec=grid_spec,
        out_shape=jax.ShapeDtypeStruct((N, 1), jnp.float32),
    )(k1, xc, w1b, b1, w2b, b2, W3, b3)


def _expand(dst, lv2d, oc):
  f = functools.partial(
      pl.kernel,
      out_type=jax.ShapeDtypeStruct((R, R), jnp.float32),
      mesh=_mesh(),
      compiler_params=_sc_params(),
      scratch_types=[
          pltpu.VMEM((ROWS_PW, R), jnp.int32),
          pltpu.VMEM((ROWS_PW, R), jnp.int32),
          pltpu.VMEM((ROWS_PW, R), jnp.float32),
          pltpu.VMEM((ROWS_PW, R), jnp.float32),
      ],
  )

  @f
  def body(dst_hbm, lv_hbm, oc_hbm, res_hbm, dst_v, lv_v, val_v, res_v):
    wid = lax.axis_index("s") * 2 + lax.axis_index("c")
    r0 = wid * ROWS_PW
    pltpu.sync_copy(dst_hbm.at[pl.ds(r0, ROWS_PW)], dst_v)
    pltpu.sync_copy(lv_hbm.at[pl.ds(r0, ROWS_PW)], lv_v)

    @pl.loop(0, ROWS_PW)
    def _rows(rr):
        @pl.loop(0, R, step=16)
        def _clamp(cs):
            dst_v[rr, pl.ds(cs, 16)] = jnp.clip(dst_v[rr, pl.ds(cs, 16)], 0,
                                                N - 1)

        pltpu.sync_copy(oc_hbm.at[dst_v.at[rr]], val_v.at[rr])

        @pl.loop(0, R, step=16)
        def _mask(cs):
            lv16 = lv_v[rr, pl.ds(cs, 16)]
            v16 = val_v[rr, pl.ds(cs, 16)]
            res_v[rr, pl.ds(cs, 16)] = jnp.where(lv16 == 0, v16, 0.0)

    pltpu.sync_copy(res_v, res_hbm.at[pl.ds(r0, ROWS_PW)])

  return body(dst, lv2d, oc)


def kernel(x, levels, W1, b1, W2, b2, W3, b3):
    lv2d = levels.astype(jnp.int32).reshape(R, R)
    w1b = W1.astype(jnp.bfloat16)
    w2b = W2.astype(jnp.bfloat16)

    dst, k1, k128 = _route(lv2d)
    idx = _scatter_idx(dst)
    xc = _gather_rows(idx, k128, x)
    oc = _mlp(k1, xc, w1b, b1, w2b, b2, W3, b3)
    res = _expand(dst, lv2d, oc.reshape(N))
    return res.reshape(N, 1)


# speedup vs baseline: 8.2333x; 8.2333x over previous
import dataclasses
import functools

import jax
import jax.numpy as jnp
from jax import lax
from jax.experimental import pallas as pl
from jax.experimental.pallas import tpu as pltpu
from jax.experimental.pallas import tpu_sc as plsc

N = 16384
D_IN = 2048
D_H = 1024
TILE_M = 256
R = 128
NW = 32
ROWS_PW = R // NW
CH = 16
NCH = N // CH
DUMP = N

def _mesh():
    return plsc.VectorSubcoreMesh(core_axis_name="c", subcore_axis_name="s")


def _sc_params():
    cp = pltpu.CompilerParams()
    if "needs_layout_passes" in pltpu.CompilerParams.__dataclass_fields__:
        cp = dataclasses.replace(cp, needs_layout_passes=False)
    return cp


def _route_body(lv_ref, dst_ref, k1_ref, k128_ref):
    lv = lv_ref[...]
    m = (lv == 0).astype(jnp.float32)
    r = lax.broadcasted_iota(jnp.int32, (R, R), 0)
    c = lax.broadcasted_iota(jnp.int32, (R, R), 1)
    upper = (r < c).astype(jnp.float32)
    lower = (c < r).astype(jnp.float32)
    e = lax.dot_general(m, upper, (((1,), (0,)), ((), ())),
                        preferred_element_type=jnp.float32)
    s = e[:, R - 1:R] + m[:, R - 1:R]
    off = lax.dot_general(lower, s, (((1,), (0,)), ((), ())),
                          preferred_element_type=jnp.float32)
    pos = (e + off + 0.5).astype(jnp.int32)
    flat = r * R + c
    dst_ref[...] = jnp.where(lv == 0, pos, DUMP + (flat - pos))
    k = jnp.sum(m).astype(jnp.int32)
    k1_ref[0] = k
    k128_ref[...] = jnp.full((R,), k, jnp.int32)


def _route(lv2d):
    return pl.pallas_call(
        _route_body,
        out_shape=(
            jax.ShapeDtypeStruct((R, R), jnp.int32),
            jax.ShapeDtypeStruct((1,), jnp.int32),
            jax.ShapeDtypeStruct((R,), jnp.int32),
        ),
        out_specs=(
            pl.BlockSpec((R, R), lambda: (0, 0)),
            pl.BlockSpec(memory_space=pltpu.SMEM),
            pl.BlockSpec((R,), lambda: (0,)),
        ),
    )(lv2d)


def _scatter_idx(dst):
  f = functools.partial(
      pl.kernel,
      out_type=jax.ShapeDtypeStruct((2 * N,), jnp.int32),
      mesh=_mesh(),
      compiler_params=_sc_params(),
      scratch_types=[
          pltpu.VMEM((ROWS_PW, R), jnp.int32),
          pltpu.VMEM((ROWS_PW, R), jnp.int32),
      ],
  )

  @f
  def body(dst_hbm, idx_hbm, dst_v, val_v):
    wid = lax.axis_index("s") * 2 + lax.axis_index("c")
    r0 = wid * ROWS_PW
    pltpu.sync_copy(dst_hbm.at[pl.ds(r0, ROWS_PW)], dst_v)

    @pl.loop(0, ROWS_PW)
    def _rows(rr):
        base = (r0 + rr) * R

        @pl.loop(0, R, step=16)
        def _cols(cs):
            val_v[rr, pl.ds(cs, 16)] = base + cs + lax.iota(jnp.int32, 16)

        pltpu.sync_copy(val_v.at[rr], idx_hbm.at[dst_v.at[rr]])

  return body(dst)


def _gather_rows(idx, k128, x):
  f = functools.partial(
      pl.kernel,
      out_type=jax.ShapeDtypeStruct((N, D_IN), jnp.float32),
      mesh=_mesh(),
      compiler_params=_sc_params(),
      scratch_types=[
          pltpu.VMEM((16,), jnp.int32),
          pltpu.VMEM((CH,), jnp.int32),
          pltpu.VMEM((CH, D_IN), jnp.float32),
      ],
  )

  @f
  def body(idx_hbm, k_hbm, x_hbm, xc_hbm, kv, iv, rows_v):
    wid = lax.axis_index("s") * 2 + lax.axis_index("c")
    pltpu.sync_copy(k_hbm.at[pl.ds(0, 16)], kv)
    k = jnp.max(kv[...], axis=0)

    @pl.loop(0, NCH // NW)
    def _chunks(ci):
        gc = ci * NW + wid

        @pl.when(gc * CH < k)
        def _():
            pltpu.sync_copy(idx_hbm.at[pl.ds(gc * CH, CH)], iv)
            iv[...] = jnp.clip(iv[...], 0, N - 1)
            pltpu.sync_copy(x_hbm.at[iv], rows_v)
            pltpu.sync_copy(rows_v, xc_hbm.at[pl.ds(gc * CH, CH)])

  return body(idx, k128, x)


def _mlp_body(k_ref, x_ref, w1_ref, b1_ref, w2_ref, b2_ref, w3_ref, b3_ref,
              o_ref):
    nt = (k_ref[0] + TILE_M - 1) // TILE_M

    @pl.when(pl.program_id(0) < nt)
    def _():
        xb = x_ref[...].astype(jnp.bfloat16)
        h1 = lax.dot_general(xb, w1_ref[...], (((1,), (0,)), ((), ())),
                             preferred_element_type=jnp.float32)
        h1 = jnp.maximum(h1 + b1_ref[...][None, :], 0.0).astype(jnp.bfloat16)
        h2 = lax.dot_general(h1, w2_ref[...], (((1,), (0,)), ((), ())),
                             preferred_element_type=jnp.float32)
        h2 = jnp.maximum(h2 + b2_ref[...][None, :], 0.0)
        out = lax.dot_general(h2, w3_ref[...], (((1,), (0,)), ((), ())),
                              preferred_element_type=jnp.float32)
        o_ref[...] = out + b3_ref[...][None, :]


def _mlp(k1, xc, w1b, b1, w2b, b2, W3, b3):
    def _x_map(i, kr):
        nt = (kr[0] + TILE_M - 1) // TILE_M
        return (jnp.minimum(i, jnp.maximum(nt - 1, 0)), 0)

    grid_spec = pltpu.PrefetchScalarGridSpec(
        num_scalar_prefetch=1,
        grid=(N // TILE_M,),
        in_specs=[
            pl.BlockSpec((TILE_M, D_IN), _x_map),
            pl.BlockSpec((D_IN, D_H), lambda i, kr: (0, 0)),
            pl.BlockSpec((D_H,), lambda i, kr: (0,)),
            pl.BlockSpec((D_H, D_H), lambda i, kr: (0, 0)),
            pl.BlockSpec((D_H,), lambda i, kr: (0,)),
            pl.BlockSpec((D_H, 1), lambda i, kr: (0, 0)),
            pl.BlockSpec((1,), lambda i, kr: (0,)),
        ],
        out_specs=pl.BlockSpec((TILE_M, 1), lambda i, kr: (i, 0)),
    )
    return pl.pallas_call(
        _mlp_body,
        grid_spec=grid_spec,
        out_shape=jax.ShapeDtypeStruct((N, 1), jnp.float32),
    )(k1, xc, w1b, b1, w2b, b2, W3, b3)


def _expand(dst, lv2d, oc):
  f = functools.partial(
      pl.kernel,
      out_type=jax.ShapeDtypeStruct((R, R), jnp.float32),
      mesh=_mesh(),
      compiler_params=_sc_params(),
      scratch_types=[
          pltpu.VMEM((ROWS_PW, R), jnp.int32),
          pltpu.VMEM((ROWS_PW, R), jnp.int32),
          pltpu.VMEM((ROWS_PW, R), jnp.float32),
          pltpu.VMEM((ROWS_PW, R), jnp.float32),
      ],
  )

  @f
  def body(dst_hbm, lv_hbm, oc_hbm, res_hbm, dst_v, lv_v, val_v, res_v):
    wid = lax.axis_index("s") * 2 + lax.axis_index("c")
    r0 = wid * ROWS_PW
    pltpu.sync_copy(dst_hbm.at[pl.ds(r0, ROWS_PW)], dst_v)
    pltpu.sync_copy(lv_hbm.at[pl.ds(r0, ROWS_PW)], lv_v)

    @pl.loop(0, ROWS_PW)
    def _rows(rr):
        @pl.loop(0, R, step=16)
        def _clamp(cs):
            dst_v[rr, pl.ds(cs, 16)] = jnp.clip(dst_v[rr, pl.ds(cs, 16)], 0,
                                                N - 1)

        pltpu.sync_copy(oc_hbm.at[dst_v.at[rr]], val_v.at[rr])

        @pl.loop(0, R, step=16)
        def _mask(cs):
            lv16 = lv_v[rr, pl.ds(cs, 16)]
            v16 = val_v[rr, pl.ds(cs, 16)]
            res_v[rr, pl.ds(cs, 16)] = jnp.where(lv16 == 0, v16, 0.0)

    pltpu.sync_copy(res_v, res_hbm.at[pl.ds(r0, ROWS_PW)])

  return body(dst, lv2d, oc)


def kernel(x, levels, W1, b1, W2, b2, W3, b3):
    lv2d = levels.astype(jnp.int32).reshape(R, R)
    w1b = W1.astype(jnp.bfloat16)
    w2b = W2.astype(jnp.bfloat16)

    dst, k1, k128 = _route(lv2d)
    idx = _scatter_idx(dst)
    xc = _gather_rows(idx, k128, x)
    oc = _mlp(k1, xc, w1b, b1, w2b, b2, W3, b3)
    res = _expand(dst, lv2d, oc.reshape(N))
    return res.reshape(N, 1)
